# Initial kernel scaffold; baseline (speedup 1.0000x reference)
import jax
import jax.numpy as jnp
from jax.experimental import pallas as pl


def kernel(x, adj, W1, b1, W2, b2):
    raise NotImplementedError("write your pallas kernel here")


# speedup vs baseline: 1.0406x; 1.0406x over previous
import functools

import jax
import jax.numpy as jnp
from jax.experimental import pallas as pl
from jax.experimental.pallas import tpu as pltpu

N = 10000
D = 128
BM = 400
NB = N // BM


def _gcn_kernel(x_ref, adj_ref, w1_ref, b1_ref, w2_ref, b2_ref,
                out_ref, y1_s, y2_s):
    l = pl.program_id(0)
    i = pl.program_id(1)

    @pl.when((l == 0) & (i == 0))
    def _init():
        y1_s[...] = jnp.dot(x_ref[...], w1_ref[...],
                            preferred_element_type=jnp.float32)

    @pl.when(l == 0)
    def _layer0():
        t = jnp.dot(adj_ref[...], y1_s[...],
                    preferred_element_type=jnp.float32)
        h = jnp.maximum(t + b1_ref[...], 0.0)
        y2_s[pl.ds(i * BM, BM), :] = jnp.dot(
            h, w2_ref[...], preferred_element_type=jnp.float32)
        out_ref[...] = h

    @pl.when(l == 1)
    def _layer1():
        t = jnp.dot(adj_ref[...], y2_s[...],
                    preferred_element_type=jnp.float32)
        out_ref[...] = jnp.maximum(t + b2_ref[...], 0.0)


@jax.jit
def kernel(x, adj, W1, b1, W2, b2):
    b1r = b1.reshape(1, D)
    b2r = b2.reshape(1, D)
    grid = (2, NB)
    return pl.pallas_call(
        _gcn_kernel,
        grid=grid,
        in_specs=[
            pl.BlockSpec((N, D), lambda l, i: (0, 0)),
            pl.BlockSpec((BM, N), lambda l, i: (i, 0)),
            pl.BlockSpec((D, D), lambda l, i: (0, 0)),
            pl.BlockSpec((1, D), lambda l, i: (0, 0)),
            pl.BlockSpec((D, D), lambda l, i: (0, 0)),
            pl.BlockSpec((1, D), lambda l, i: (0, 0)),
        ],
        out_specs=pl.BlockSpec((BM, D), lambda l, i: (i, 0)),
        out_shape=jax.ShapeDtypeStruct((N, D), jnp.float32),
        scratch_shapes=[
            pltpu.VMEM((N, D), jnp.float32),
            pltpu.VMEM((N, D), jnp.float32),
        ],
        compiler_params=pltpu.CompilerParams(
            dimension_semantics=("arbitrary", "arbitrary"),
            vmem_limit_bytes=110 * 1024 * 1024,
        ),
    )(x, adj, W1, b1r, W2, b2r)
